# baseline (speedup 1.0000x reference)
import jax
import jax.numpy as jnp
from jax.experimental import pallas as pl
from jax.experimental.pallas import tpu as pltpu

_NBINS = 51
_BW = 2.0 / (_NBINS - 1)
_INV_BW = 1.0 / _BW
_BS = 1024
_D = 128
_T = 128
_NT = _BS // _T
_NALL = _BS * (_BS - 1) / 2
_CNT_LANE = 4092.0
_CNT_ROW = 51
_ACC_ROWS = 56


def _rowsum(c_bf16, ones_bf):
    return jax.lax.dot_general(ones_bf, c_bf16, (((1,), (0,)), ((), ())),
                               preferred_element_type=jnp.float32)


def _sweep(sp, evm_bf, vm_bf, ones_bf, ap_ref, aa_ref):
    for t in range(_NBINS):
        d = (sp - float(t)).astype(jnp.bfloat16)
        c = jnp.minimum(jnp.maximum(d, 0.0), 1.0)
        ca = c if vm_bf is None else c * vm_bf
        cp = c * evm_bf
        aa_ref[t:t + 1, :] = aa_ref[t:t + 1, :] + _rowsum(ca, ones_bf)
        ap_ref[t:t + 1, :] = ap_ref[t:t + 1, :] + _rowsum(cp, ones_bf)
    ap_ref[_CNT_ROW:_CNT_ROW + 1, :] = (
        ap_ref[_CNT_ROW:_CNT_ROW + 1, :] + _rowsum(evm_bf, ones_bf))


def _body(x_ref, lr_ref, lc_ref, out_ref, ap_ref, aa_ref):
    ap_ref[...] = jnp.zeros_like(ap_ref)
    aa_ref[...] = jnp.zeros_like(aa_ref)

    ii = jax.lax.broadcasted_iota(jnp.int32, (_T, _T), 0)
    jj = jax.lax.broadcasted_iota(jnp.int32, (_T, _T), 1)
    diag_vm_f = jnp.where(ii == jj, 0.0, 0.5)
    diag_vm = diag_vm_f.astype(jnp.bfloat16)
    ones_bf = jnp.ones((1, _T), jnp.bfloat16)
    dn = (((1,), (1,)), ((), ()))

    for rb in range(_NT):
        xr = x_ref[rb * _T:(rb + 1) * _T, :]
        lr = lr_ref[rb * _T:(rb + 1) * _T, :]
        for cb in range(rb, _NT):
            xc = x_ref[cb * _T:(cb + 1) * _T, :]
            s = jax.lax.dot_general(xr, xc, dn,
                                    preferred_element_type=jnp.float32)
            sp = s * _INV_BW + _INV_BW
            eq = lr == lc_ref[cb]
            if rb == cb:
                evm_bf = jnp.where(eq, diag_vm_f, 0.0).astype(jnp.bfloat16)
                _sweep(sp, evm_bf, diag_vm, ones_bf, ap_ref, aa_ref)
            else:
                evm_bf = jnp.where(eq, 1.0, 0.0).astype(jnp.bfloat16)
                _sweep(sp, evm_bf, None, ones_bf, ap_ref, aa_ref)

    qa = aa_ref[0:_NBINS, :]
    qp = ap_ref[0:_NBINS, :]
    ha2 = jnp.concatenate(
        [_CNT_LANE - qa[0:1, :], qa[0:_NBINS - 1, :] - qa[1:_NBINS, :]],
        axis=0)
    hp2 = jnp.concatenate(
        [ap_ref[_CNT_ROW:_CNT_ROW + 1, :] - qp[0:1, :],
         qp[0:_NBINS - 1, :] - qp[1:_NBINS, :]],
        axis=0)
    cnt2 = ap_ref[_CNT_ROW:_CNT_ROW + 1, :]

    ones = jnp.ones((1, _T), jnp.float32)
    dnl = (((1,), (1,)), ((), ()))
    ha = jax.lax.dot_general(ones, ha2, dnl,
                             preferred_element_type=jnp.float32)
    hp = jax.lax.dot_general(ones, hp2, dnl,
                             preferred_element_type=jnp.float32)
    npos = jax.lax.dot_general(ones, cnt2, dnl,
                               preferred_element_type=jnp.float32)
    nneg = _NALL - npos

    hist_pos = hp / npos
    hist_neg = (ha - hp) / nneg

    m_i = jax.lax.broadcasted_iota(jnp.int32, (_NBINS, _NBINS), 0)
    k_i = jax.lax.broadcasted_iota(jnp.int32, (_NBINS, _NBINS), 1)
    tri = (m_i <= k_i).astype(jnp.float32)
    cdf = jnp.dot(hist_pos, tri, preferred_element_type=jnp.float32)

    out_ref[...] = jnp.sum(hist_neg * cdf, axis=1, keepdims=True)


def kernel(x, labels):
    lab = labels.astype(jnp.int32)
    lab_row = lab.reshape(_BS, 1)
    lab_col = lab.reshape(_NT, 1, _T)

    loss = pl.pallas_call(
        _body,
        scratch_shapes=[
            pltpu.VMEM((_ACC_ROWS, _T), jnp.float32),
            pltpu.VMEM((_ACC_ROWS, _T), jnp.float32),
        ],
        out_shape=jax.ShapeDtypeStruct((1, 1), jnp.float32),
    )(x, lab_row, lab_col)
    return loss[0, 0]


# speedup vs baseline: 1.3910x; 1.3910x over previous
import jax
import jax.numpy as jnp
from jax.experimental import pallas as pl
from jax.experimental.pallas import tpu as pltpu

_NBINS = 51
_BW = 2.0 / (_NBINS - 1)
_INV_BW = 1.0 / _BW
_BS = 1024
_D = 128
_T = 128
_NT = _BS // _T
_NALL = _BS * (_BS - 1) / 2
_CNT_LANE = 4092.0
_GROUPS = (8, 8, 8, 8, 8, 8, 3)
_CNT_GROW = 7


def _sweep(sp, evm_bf, vm_bf, ones_bf, ap_ref, aa_ref):
    dn = (((1,), (0,)), ((), ()))
    t0 = 0
    for g, gw in enumerate(_GROUPS):
        pa, pp = [], []
        for t in range(t0, t0 + gw):
            d = (sp - float(t)).astype(jnp.bfloat16)
            c = jnp.minimum(jnp.maximum(d, 0.0), 1.0)
            pa.append(c if vm_bf is None else c * vm_bf)
            pp.append(c * evm_bf)
        t0 += gw
        ca = jnp.concatenate(pa, axis=1)
        cp = jnp.concatenate(pp, axis=1)
        w = _T * gw
        ra = jax.lax.dot_general(ones_bf, ca, dn,
                                 preferred_element_type=jnp.float32)
        rp = jax.lax.dot_general(ones_bf, cp, dn,
                                 preferred_element_type=jnp.float32)
        aa_ref[g:g + 1, 0:w] = aa_ref[g:g + 1, 0:w] + ra
        ap_ref[g:g + 1, 0:w] = ap_ref[g:g + 1, 0:w] + rp
    rc = jax.lax.dot_general(ones_bf, evm_bf, dn,
                             preferred_element_type=jnp.float32)
    ap_ref[_CNT_GROW:_CNT_GROW + 1, 0:_T] = (
        ap_ref[_CNT_GROW:_CNT_GROW + 1, 0:_T] + rc)


def _hist_body(x_ref, lr_ref, lc_ref, ap_ref, aa_ref):
    ap_ref[...] = jnp.zeros_like(ap_ref)
    aa_ref[...] = jnp.zeros_like(aa_ref)

    ii = jax.lax.broadcasted_iota(jnp.int32, (_T, _T), 0)
    jj = jax.lax.broadcasted_iota(jnp.int32, (_T, _T), 1)
    diag_vm_f = jnp.where(ii == jj, 0.0, 0.5)
    diag_vm = diag_vm_f.astype(jnp.bfloat16)
    ones_bf = jnp.ones((1, _T), jnp.bfloat16)
    dn = (((1,), (1,)), ((), ()))

    for rb in range(_NT):
        xr = x_ref[rb * _T:(rb + 1) * _T, :]
        lr = lr_ref[rb * _T:(rb + 1) * _T, :]
        for cb in range(rb, _NT):
            xc = x_ref[cb * _T:(cb + 1) * _T, :]
            s = jax.lax.dot_general(xr, xc, dn,
                                    preferred_element_type=jnp.float32)
            sp = s * _INV_BW + _INV_BW
            eq = lr == lc_ref[cb]
            if rb == cb:
                evm_bf = jnp.where(eq, diag_vm_f, 0.0).astype(jnp.bfloat16)
                _sweep(sp, evm_bf, diag_vm, ones_bf, ap_ref, aa_ref)
            else:
                evm_bf = jnp.where(eq, 1.0, 0.0).astype(jnp.bfloat16)
                _sweep(sp, evm_bf, None, ones_bf, ap_ref, aa_ref)


def _finalize_body(qp_ref, qa_ref, out_ref):
    qa = qa_ref[0:_NBINS, :]
    qp = qp_ref[0:_NBINS, :]
    cntp = qp_ref[56:57, :]
    ha2 = jnp.concatenate(
        [_CNT_LANE - qa[0:1, :], qa[0:_NBINS - 1, :] - qa[1:_NBINS, :]],
        axis=0)
    hp2 = jnp.concatenate(
        [cntp - qp[0:1, :], qp[0:_NBINS - 1, :] - qp[1:_NBINS, :]],
        axis=0)

    ones = jnp.ones((1, _T), jnp.float32)
    dnl = (((1,), (1,)), ((), ()))
    ha = jax.lax.dot_general(ones, ha2, dnl,
                             preferred_element_type=jnp.float32)
    hp = jax.lax.dot_general(ones, hp2, dnl,
                             preferred_element_type=jnp.float32)
    npos = jax.lax.dot_general(ones, cntp, dnl,
                               preferred_element_type=jnp.float32)
    nneg = _NALL - npos

    hist_pos = hp / npos
    hist_neg = (ha - hp) / nneg

    m_i = jax.lax.broadcasted_iota(jnp.int32, (_NBINS, _NBINS), 0)
    k_i = jax.lax.broadcasted_iota(jnp.int32, (_NBINS, _NBINS), 1)
    tri = (m_i <= k_i).astype(jnp.float32)
    cdf = jnp.dot(hist_pos, tri, preferred_element_type=jnp.float32)

    out_ref[...] = jnp.sum(hist_neg * cdf, axis=1, keepdims=True)


def kernel(x, labels):
    lab = labels.astype(jnp.int32)
    lab_row = lab.reshape(_BS, 1)
    lab_col = lab.reshape(_NT, 1, _T)

    ap_acc, aa_acc = pl.pallas_call(
        _hist_body,
        out_shape=[
            jax.ShapeDtypeStruct((8, 8 * _T), jnp.float32),
            jax.ShapeDtypeStruct((8, 8 * _T), jnp.float32),
        ],
    )(x, lab_row, lab_col)

    loss = pl.pallas_call(
        _finalize_body,
        out_shape=jax.ShapeDtypeStruct((1, 1), jnp.float32),
    )(ap_acc.reshape(64, _T), aa_acc.reshape(64, _T))
    return loss[0, 0]
